# baseline (speedup 1.0000x reference)
import jax
import jax.numpy as jnp
from jax import lax
from jax.experimental import pallas as pl
from jax.experimental.pallas import tpu as pltpu
from jax.experimental.pallas import tpu_sc as plsc

_N = 131072
_K = 512
_D = 32
_BETA = 0.25
_BN = 2048
_G = _N // _BN

_NC = 2
_NS = 16
_NW = _NC * _NS
_BPW = _N // _NW
_CH = 2048
_NCH = _BPW // _CH


def _dist_body(z_ref, emb_ref, inds_ref, loss_ref):
    i = pl.program_id(0)
    z = z_ref[...]
    emb = emb_ref[...]
    e_sq = jnp.sum(emb * emb, axis=1)
    z_sq = jnp.sum(z * z, axis=1)
    dist = (z_sq[:, None] + e_sq[None, :]) - 2.0 * jax.lax.dot_general(
        z.astype(jnp.bfloat16), emb.astype(jnp.bfloat16),
        (((1,), (1,)), ((), ())),
        preferred_element_type=jnp.float32)
    col = jax.lax.broadcasted_iota(jnp.int32, (_BN, _K), 1)
    dmin = jnp.min(dist, axis=1)
    inds = jnp.min(jnp.where(dist == dmin[:, None], col, _K),
                   axis=1).astype(jnp.int32)
    inds_ref[0, 0, :] = inds
    partial = jnp.sum(dmin).reshape(1, 1)

    @pl.when(i == 0)
    def _():
        loss_ref[...] = jnp.zeros((1, 1), jnp.float32)

    loss_ref[...] += partial


def _gather_body(emb_hbm, idx_hbm, out_hbm, idx_v, rows_v, sem):
    wid = lax.axis_index("s") * _NC + lax.axis_index("c")
    base = wid * _BPW
    for c in range(_NCH):
        off = base + c * _CH
        pltpu.sync_copy(idx_hbm.at[pl.ds(off, _CH)], idx_v)
        pltpu.async_copy(emb_hbm.at[idx_v], rows_v, sem).wait()
        pltpu.sync_copy(rows_v, out_hbm.at[pl.ds(off, _CH)])


_sc_gather = pl.kernel(
    _gather_body,
    out_type=jax.ShapeDtypeStruct((_N, _D), jnp.float32),
    mesh=plsc.VectorSubcoreMesh(core_axis_name="c", subcore_axis_name="s"),
    scratch_types=[
        pltpu.VMEM((_CH,), jnp.int32),
        pltpu.VMEM((_CH, _D), jnp.float32),
        pltpu.SemaphoreType.DMA,
    ],
    compiler_params=pltpu.CompilerParams(use_tc_tiling_on_sc=False),
)


def kernel(z_e, embeddings):
    inds3, loss_acc = pl.pallas_call(
        _dist_body,
        grid=(_G,),
        in_specs=[
            pl.BlockSpec((_BN, _D), lambda i: (i, 0)),
            pl.BlockSpec((_K, _D), lambda i: (0, 0)),
        ],
        out_specs=[
            pl.BlockSpec((1, 1, _BN), lambda i: (i, 0, 0)),
            pl.BlockSpec((1, 1), lambda i: (0, 0)),
        ],
        out_shape=[
            jax.ShapeDtypeStruct((_G, 1, _BN), jnp.int32),
            jax.ShapeDtypeStruct((1, 1), jnp.float32),
        ],
    )(z_e, embeddings)
    inds = inds3.reshape(_N)
    zq = _sc_gather(embeddings, inds)
    loss = loss_acc[0, 0] * ((1.0 + _BETA) / (_N * _D))
    return (zq, inds, loss)


# speedup vs baseline: 1.1361x; 1.1361x over previous
import jax
import jax.numpy as jnp
from jax import lax
from jax.experimental import pallas as pl
from jax.experimental.pallas import tpu as pltpu
from jax.experimental.pallas import tpu_sc as plsc

_N = 131072
_K = 512
_D = 32
_BETA = 0.25
_BN = 2048
_G = _N // _BN

_NC = 2
_NS = 16
_NW = _NC * _NS
_BPW = _N // _NW
_CH = 2048
_NCH = _BPW // _CH


def _dist_body(z_ref, emb_ref, inds_ref, loss_ref):
    i = pl.program_id(0)
    z = z_ref[...]
    emb = emb_ref[...]
    e_sq = jnp.sum(emb * emb, axis=1, keepdims=True)
    z_sq = jnp.sum(z * z, axis=1, keepdims=True)
    z_sq_row = jax.lax.transpose(z_sq, (1, 0))
    dist = (z_sq_row + e_sq) - 2.0 * jax.lax.dot_general(
        emb.astype(jnp.bfloat16), z.astype(jnp.bfloat16),
        (((1,), (1,)), ((), ())),
        preferred_element_type=jnp.float32)
    row = jax.lax.broadcasted_iota(jnp.int32, (_K, _BN), 0)
    dmin = jnp.min(dist, axis=0)
    inds = jnp.min(jnp.where(dist == dmin[None, :], row, _K),
                   axis=0).astype(jnp.int32)
    inds_ref[0, 0, :] = inds
    partial = jnp.sum(dmin).reshape(1, 1)

    @pl.when(i == 0)
    def _():
        loss_ref[...] = jnp.zeros((1, 1), jnp.float32)

    loss_ref[...] += partial


def _gather_body(emb_hbm, idx_hbm, out_hbm, idx_v, rows_v, sem):
    wid = lax.axis_index("s") * _NC + lax.axis_index("c")
    base = wid * _BPW
    for c in range(_NCH):
        off = base + c * _CH
        pltpu.sync_copy(idx_hbm.at[pl.ds(off, _CH)], idx_v)
        pltpu.async_copy(emb_hbm.at[idx_v], rows_v, sem).wait()
        pltpu.sync_copy(rows_v, out_hbm.at[pl.ds(off, _CH)])


_sc_gather = pl.kernel(
    _gather_body,
    out_type=jax.ShapeDtypeStruct((_N, _D), jnp.float32),
    mesh=plsc.VectorSubcoreMesh(core_axis_name="c", subcore_axis_name="s"),
    scratch_types=[
        pltpu.VMEM((_CH,), jnp.int32),
        pltpu.VMEM((_CH, _D), jnp.float32),
        pltpu.SemaphoreType.DMA,
    ],
    compiler_params=pltpu.CompilerParams(use_tc_tiling_on_sc=False),
)


def kernel(z_e, embeddings):
    inds3, loss_acc = pl.pallas_call(
        _dist_body,
        grid=(_G,),
        in_specs=[
            pl.BlockSpec((_BN, _D), lambda i: (i, 0)),
            pl.BlockSpec((_K, _D), lambda i: (0, 0)),
        ],
        out_specs=[
            pl.BlockSpec((1, 1, _BN), lambda i: (i, 0, 0)),
            pl.BlockSpec((1, 1), lambda i: (0, 0)),
        ],
        out_shape=[
            jax.ShapeDtypeStruct((_G, 1, _BN), jnp.int32),
            jax.ShapeDtypeStruct((1, 1), jnp.float32),
        ],
    )(z_e, embeddings)
    inds = inds3.reshape(_N)
    zq = _sc_gather(embeddings, inds)
    loss = loss_acc[0, 0] * ((1.0 + _BETA) / (_N * _D))
    return (zq, inds, loss)


# speedup vs baseline: 1.1592x; 1.0203x over previous
import jax
import jax.numpy as jnp
from jax import lax
from jax.experimental import pallas as pl
from jax.experimental.pallas import tpu as pltpu
from jax.experimental.pallas import tpu_sc as plsc

_N = 131072
_K = 512
_D = 32
_BETA = 0.25
_BN = 4096
_G = _N // _BN

_NC = 2
_NS = 16
_NW = _NC * _NS
_BPW = _N // _NW
_CH = 2048
_NCH = _BPW // _CH


def _dist_body(z_ref, emb_ref, inds_ref, loss_ref):
    i = pl.program_id(0)
    z = z_ref[...]
    emb = emb_ref[...]
    e_sq = jnp.sum(emb * emb, axis=1, keepdims=True)
    z_sq = jnp.sum(z * z, axis=1, keepdims=True)
    z_sq_row = jax.lax.transpose(z_sq, (1, 0))
    dist = (z_sq_row + e_sq) - 2.0 * jax.lax.dot_general(
        emb.astype(jnp.bfloat16), z.astype(jnp.bfloat16),
        (((1,), (1,)), ((), ())),
        preferred_element_type=jnp.float32)
    row = jax.lax.broadcasted_iota(jnp.int32, (_K, _BN), 0)
    dmin = jnp.min(dist, axis=0)
    inds = jnp.min(jnp.where(dist == dmin[None, :], row, _K),
                   axis=0).astype(jnp.int32)
    inds_ref[0, 0, :] = inds
    partial = jnp.sum(dmin).reshape(1, 1)

    @pl.when(i == 0)
    def _():
        loss_ref[...] = jnp.zeros((1, 1), jnp.float32)

    loss_ref[...] += partial


def _gather_body(emb_hbm, idx_hbm, out_hbm, idx_v, rows_v, sem):
    wid = lax.axis_index("s") * _NC + lax.axis_index("c")
    base = wid * _BPW
    for c in range(_NCH):
        off = base + c * _CH
        pltpu.sync_copy(idx_hbm.at[pl.ds(off, _CH)], idx_v)
        pltpu.async_copy(emb_hbm.at[idx_v], rows_v, sem).wait()
        pltpu.sync_copy(rows_v, out_hbm.at[pl.ds(off, _CH)])


_sc_gather = pl.kernel(
    _gather_body,
    out_type=jax.ShapeDtypeStruct((_N, _D), jnp.float32),
    mesh=plsc.VectorSubcoreMesh(core_axis_name="c", subcore_axis_name="s"),
    scratch_types=[
        pltpu.VMEM((_CH,), jnp.int32),
        pltpu.VMEM((_CH, _D), jnp.float32),
        pltpu.SemaphoreType.DMA,
    ],
    compiler_params=pltpu.CompilerParams(use_tc_tiling_on_sc=False),
)


def kernel(z_e, embeddings):
    inds3, loss_acc = pl.pallas_call(
        _dist_body,
        grid=(_G,),
        in_specs=[
            pl.BlockSpec((_BN, _D), lambda i: (i, 0)),
            pl.BlockSpec((_K, _D), lambda i: (0, 0)),
        ],
        out_specs=[
            pl.BlockSpec((1, 1, _BN), lambda i: (i, 0, 0)),
            pl.BlockSpec((1, 1), lambda i: (0, 0)),
        ],
        out_shape=[
            jax.ShapeDtypeStruct((_G, 1, _BN), jnp.int32),
            jax.ShapeDtypeStruct((1, 1), jnp.float32),
        ],
    )(z_e, embeddings)
    inds = inds3.reshape(_N)
    zq = _sc_gather(embeddings, inds)
    loss = loss_acc[0, 0] * ((1.0 + _BETA) / (_N * _D))
    return (zq, inds, loss)
